# baseline (speedup 1.0000x reference)
import jax
import jax.numpy as jnp
from jax import lax
from jax.experimental import pallas as pl
from jax.experimental.pallas import tpu as pltpu
from jax.experimental.pallas import tpu_sc as plsc

N = 10000
K = 16
C = 128
H = 8
HC = C // H
WS = 0.16
QS = 0.04
QGL = int((2 * WS + 1e-4) // QS)
L = 2 * QGL
SCALE = HC ** -0.5
HID = 4 * C

NP = 10240
PP = NP * K

GD1 = 3 * C
GD2 = 2 * C


ROWS_A = 1024


def _ln(x, s, b):
    mu = jnp.mean(x, axis=-1, keepdims=True)
    xc = x - mu
    var = jnp.mean(xc * xc, axis=-1, keepdims=True)
    return xc / jnp.sqrt(var + 1e-5) * s + b


def _ln_qkv_body(x_ref, c_ref, s_ref, b_ref, w_ref, bias_ref, qs_ref, kvc_ref):
    h = _ln(x_ref[...], s_ref[...], b_ref[...])
    qkv = jnp.dot(h, w_ref[...], preferred_element_type=jnp.float32) + bias_ref[...]
    qs_ref[...] = qkv[:, :C] * SCALE
    kvc_ref[:, : 2 * C] = qkv[:, C:]
    kvc_ref[:, 2 * C :] = c_ref[...]


def _ln_qkv(feats, coords128, ln_s, ln_b, w_qkv, b_qkv):
    grid = (NP // ROWS_A,)
    return pl.pallas_call(
        _ln_qkv_body,
        grid=grid,
        in_specs=[
            pl.BlockSpec((ROWS_A, C), lambda i: (i, 0)),
            pl.BlockSpec((ROWS_A, C), lambda i: (i, 0)),
            pl.BlockSpec((1, C), lambda i: (0, 0)),
            pl.BlockSpec((1, C), lambda i: (0, 0)),
            pl.BlockSpec((C, 3 * C), lambda i: (0, 0)),
            pl.BlockSpec((1, 3 * C), lambda i: (0, 0)),
        ],
        out_specs=[
            pl.BlockSpec((ROWS_A, C), lambda i: (i, 0)),
            pl.BlockSpec((ROWS_A, GD1), lambda i: (i, 0)),
        ],
        out_shape=[
            jax.ShapeDtypeStruct((NP, C), jnp.float32),
            jax.ShapeDtypeStruct((NP, GD1), jnp.float32),
        ],
    )(feats, coords128, ln_s, ln_b, w_qkv, b_qkv)


CHUNK = 128
NC = 2
NS = 16
NW = NC * NS
NCHUNKS = PP // CHUNK
ITERS = NCHUNKS // NW
OUTER = ITERS // 2


def _gather_pairs(kvc, i1t, width):
    mesh = plsc.VectorSubcoreMesh(
        core_axis_name="c", subcore_axis_name="s", num_cores=NC, num_subcores=NS
    )

    def body(kvc_hbm, idx_hbm, out_hbm,
             idx_a, idx_b, rows_a, rows_b, sg_a, sg_b):
        wid = lax.axis_index("s") * NC + lax.axis_index("c")
        slots = ((idx_a, rows_a, sg_a), (idx_b, rows_b, sg_b))

        def base(j):
            return (j * NW + wid) * CHUNK

        def step(j, carry):
            idx_v, rows_v, sg = slots[0]
            base_j = base(j)
            pltpu.sync_copy(idx_hbm.at[pl.ds(base_j, CHUNK)], idx_v)
            pltpu.async_copy(kvc_hbm.at[idx_v], rows_v, sg).wait()
            pltpu.sync_copy(rows_v, out_hbm.at[pl.ds(base_j, CHUNK)])
            return carry

        lax.fori_loop(0, ITERS, step, 0)

    f = pl.kernel(
        body,
        out_type=jax.ShapeDtypeStruct((PP, width), jnp.float32),
        mesh=mesh,
        scratch_types=[
            pltpu.VMEM((CHUNK,), jnp.int32),
            pltpu.VMEM((CHUNK,), jnp.int32),
            pltpu.VMEM((CHUNK, width), jnp.float32),
            pltpu.VMEM((CHUNK, width), jnp.float32),
            pltpu.SemaphoreType.DMA,
            pltpu.SemaphoreType.DMA,
        ],
    )
    return f(kvc, i1t)


TQ = 256
RP = TQ * K
NT = NP // TQ


def _head_matrices():
    lane = jnp.arange(C, dtype=jnp.int32)[:, None]
    head = jnp.arange(H, dtype=jnp.int32)[None, :]
    eh = (lane // HC == head).astype(jnp.float32)
    return eh, eh.T


def _attn_math(kg, vg, rid3, q, resid, t_ref, eh_ref, r_ref, w_ref, b_ref):
    cols = lax.broadcasted_iota(jnp.int32, (K, TQ, 3 * L), 2)
    oh = (
        (cols == (rid3 & (L - 1)))
        | (cols == ((rid3 >> 4) & (L - 1)) + L)
        | (cols == (rid3 >> 8) + 2 * L)
    ).astype(jnp.float32)
    G = jnp.dot(
        oh.reshape(RP, 3 * L), t_ref[...], preferred_element_type=jnp.float32
    ).reshape(K, TQ, 3 * C)

    qg = q[None]
    s = qg * (kg + G[:, :, :C]) + kg * G[:, :, C : 2 * C]
    attn = jnp.dot(
        s.reshape(RP, C), eh_ref[...], preferred_element_type=jnp.float32
    ).reshape(K, TQ, H)

    m = jnp.max(attn, axis=0, keepdims=True)
    e = jnp.exp(attn - m)
    den = jnp.sum(e, axis=0, keepdims=True)
    p = e / den

    pb = jnp.dot(
        p.reshape(RP, H), r_ref[...], preferred_element_type=jnp.float32
    ).reshape(K, TQ, C)
    vt = vg + G[:, :, 2 * C :]
    o = jnp.sum(pb * vt, axis=0)

    x = jnp.dot(o, w_ref[...], preferred_element_type=jnp.float32)
    return x + b_ref[...] + resid


def _attn1_body(g_ref, q_ref, c_ref, res_ref, t_ref, eh_ref, r_ref, w_ref,
                b_ref, o_ref, rid_ref):
    g = g_ref[...]
    cg = g[:, :, 2 * C : 2 * C + 3]
    cq = c_ref[...][None, :, :3]
    rel = cq - cg
    rel = jnp.round(rel * 100000.0) / 100000.0
    rpi = jnp.floor((rel + 2 * WS - 1e-4) / QS).astype(jnp.int32)
    rpi = jnp.clip(rpi, 0, L - 1)
    rid3 = (
        rpi[:, :, 0:1] + rpi[:, :, 1:2] * L + rpi[:, :, 2:3] * (L * L)
    )
    rid_ref[...] = rid3.reshape(1, K, TQ)

    o_ref[...] = _attn_math(
        g[:, :, :C], g[:, :, C : 2 * C], rid3, q_ref[...], res_ref[...],
        t_ref, eh_ref, r_ref, w_ref, b_ref,
    )


def _attn2_body(g_ref, q_ref, rid_ref, res_ref, t_ref, eh_ref, r_ref, w_ref,
                b_ref, o_ref):
    g = g_ref[...]
    rid3 = rid_ref[...].reshape(K, TQ, 1)
    o_ref[...] = _attn_math(
        g[:, :, :C], g[:, :, C : 2 * C], rid3, q_ref[...], res_ref[...],
        t_ref, eh_ref, r_ref, w_ref, b_ref,
    )


def _attention1(g3, qs, coords16, resid, t_stack, eh, r, w_proj, b_proj):
    return pl.pallas_call(
        _attn1_body,
        grid=(NT,),
        in_specs=[
            pl.BlockSpec((K, TQ, GD1), lambda i: (0, i, 0)),
            pl.BlockSpec((TQ, C), lambda i: (i, 0)),
            pl.BlockSpec((TQ, 16), lambda i: (i, 0)),
            pl.BlockSpec((TQ, C), lambda i: (i, 0)),
            pl.BlockSpec((3 * L, 3 * C), lambda i: (0, 0)),
            pl.BlockSpec((C, H), lambda i: (0, 0)),
            pl.BlockSpec((H, C), lambda i: (0, 0)),
            pl.BlockSpec((C, C), lambda i: (0, 0)),
            pl.BlockSpec((1, C), lambda i: (0, 0)),
        ],
        out_specs=[
            pl.BlockSpec((TQ, C), lambda i: (i, 0)),
            pl.BlockSpec((1, K, TQ), lambda i: (i, 0, 0)),
        ],
        out_shape=[
            jax.ShapeDtypeStruct((NP, C), jnp.float32),
            jax.ShapeDtypeStruct((NT, K, TQ), jnp.int32),
        ],
    )(g3, qs, coords16, resid, t_stack, eh, r, w_proj, b_proj)


def _attention2(g3, qs, rid, resid, t_stack, eh, r, w_proj, b_proj):
    return pl.pallas_call(
        _attn2_body,
        grid=(NT,),
        in_specs=[
            pl.BlockSpec((K, TQ, GD2), lambda i: (0, i, 0)),
            pl.BlockSpec((TQ, C), lambda i: (i, 0)),
            pl.BlockSpec((1, K, TQ), lambda i: (i, 0, 0)),
            pl.BlockSpec((TQ, C), lambda i: (i, 0)),
            pl.BlockSpec((3 * L, 3 * C), lambda i: (0, 0)),
            pl.BlockSpec((C, H), lambda i: (0, 0)),
            pl.BlockSpec((H, C), lambda i: (0, 0)),
            pl.BlockSpec((C, C), lambda i: (0, 0)),
            pl.BlockSpec((1, C), lambda i: (0, 0)),
        ],
        out_specs=pl.BlockSpec((TQ, C), lambda i: (i, 0)),
        out_shape=jax.ShapeDtypeStruct((NP, C), jnp.float32),
    )(g3, qs, rid, resid, t_stack, eh, r, w_proj, b_proj)


ROWS_D = 1024


def _mlp_out(x, s_ref, b_ref, w1_ref, b1_ref, w2_ref, b2_ref):
    h = _ln(x, s_ref[...], b_ref[...])
    f = jax.nn.gelu(jnp.dot(h, w1_ref[...], preferred_element_type=jnp.float32) + b1_ref[...])
    return x + jnp.dot(f, w2_ref[...], preferred_element_type=jnp.float32) + b2_ref[...]


def _mlp_body(x_ref, s_ref, b_ref, w1_ref, b1_ref, w2_ref, b2_ref, o_ref):
    o_ref[...] = _mlp_out(x_ref[...], s_ref, b_ref, w1_ref, b1_ref, w2_ref, b2_ref)


def _mlp_qkv_body(x_ref, s_ref, b_ref, w1_ref, b1_ref, w2_ref, b2_ref,
                  s2_ref, bb2_ref, wq_ref, bq_ref, o_ref, qs_ref, kv_ref):
    y = _mlp_out(x_ref[...], s_ref, b_ref, w1_ref, b1_ref, w2_ref, b2_ref)
    o_ref[...] = y
    h = _ln(y, s2_ref[...], bb2_ref[...])
    qkv = jnp.dot(h, wq_ref[...], preferred_element_type=jnp.float32) + bq_ref[...]
    qs_ref[...] = qkv[:, :C] * SCALE
    kv_ref[...] = qkv[:, C:]


def _mlp(x, ln_s, ln_b, w1, b1, w2, b2):
    wspec = [
        pl.BlockSpec((1, C), lambda i: (0, 0)),
        pl.BlockSpec((1, C), lambda i: (0, 0)),
        pl.BlockSpec((C, HID), lambda i: (0, 0)),
        pl.BlockSpec((1, HID), lambda i: (0, 0)),
        pl.BlockSpec((HID, C), lambda i: (0, 0)),
        pl.BlockSpec((1, C), lambda i: (0, 0)),
    ]
    return pl.pallas_call(
        _mlp_body,
        grid=(NP // ROWS_D,),
        in_specs=[pl.BlockSpec((ROWS_D, C), lambda i: (i, 0))] + wspec,
        out_specs=pl.BlockSpec((ROWS_D, C), lambda i: (i, 0)),
        out_shape=jax.ShapeDtypeStruct((NP, C), jnp.float32),
    )(x, ln_s, ln_b, w1, b1, w2, b2)


def _mlp_qkv(x, ln_s, ln_b, w1, b1, w2, b2, ln2_s, ln2_b, w_qkv, b_qkv):
    wspec = [
        pl.BlockSpec((1, C), lambda i: (0, 0)),
        pl.BlockSpec((1, C), lambda i: (0, 0)),
        pl.BlockSpec((C, HID), lambda i: (0, 0)),
        pl.BlockSpec((1, HID), lambda i: (0, 0)),
        pl.BlockSpec((HID, C), lambda i: (0, 0)),
        pl.BlockSpec((1, C), lambda i: (0, 0)),
        pl.BlockSpec((1, C), lambda i: (0, 0)),
        pl.BlockSpec((1, C), lambda i: (0, 0)),
        pl.BlockSpec((C, 3 * C), lambda i: (0, 0)),
        pl.BlockSpec((1, 3 * C), lambda i: (0, 0)),
    ]
    return pl.pallas_call(
        _mlp_qkv_body,
        grid=(NP // ROWS_D,),
        in_specs=[pl.BlockSpec((ROWS_D, C), lambda i: (i, 0))] + wspec,
        out_specs=[
            pl.BlockSpec((ROWS_D, C), lambda i: (i, 0)),
            pl.BlockSpec((ROWS_D, C), lambda i: (i, 0)),
            pl.BlockSpec((ROWS_D, GD2), lambda i: (i, 0)),
        ],
        out_shape=[
            jax.ShapeDtypeStruct((NP, C), jnp.float32),
            jax.ShapeDtypeStruct((NP, C), jnp.float32),
            jax.ShapeDtypeStruct((NP, GD2), jnp.float32),
        ],
    )(x, ln_s, ln_b, w1, b1, w2, b2, ln2_s, ln2_b, w_qkv, b_qkv)


def _stack_tables(p):
    parts = []
    for name in ("tab_q", "tab_k", "tab_v"):
        t = p[name]
        parts.append(jnp.concatenate([t[:, :, :, d].reshape(L, C) for d in range(3)], axis=0))
    return jnp.concatenate(parts, axis=1)


def kernel(feats, coords, params, index_0, index_1, index_0_offsets, n_max):
    feats_p = jnp.pad(feats, ((0, NP - N), (0, 0)))
    coords16 = jnp.pad(coords, ((0, NP - N), (0, 13)))
    coords128 = jnp.pad(coords, ((0, NP - N), (0, C - 3)))
    p1, p2 = params["blocks"]
    t1 = _stack_tables(p1)
    t2 = _stack_tables(p2)
    eh, r = _head_matrices()
    i1t = jnp.pad(index_1.reshape(N, K), ((0, NP - N), (0, 0))).T.reshape(PP)

    qs, kvc = _ln_qkv(
        feats_p, coords128,
        p1["ln1_s"].reshape(1, C), p1["ln1_b"].reshape(1, C),
        p1["w_qkv"], p1["b_qkv"].reshape(1, 3 * C),
    )
    g1 = _gather_pairs(kvc, i1t, GD1).reshape(K, NP, GD1)
    x, rid = _attention1(g1, qs, coords16, feats_p, t1, eh, r, p1["w_proj"], p1["b_proj"].reshape(1, C))
    x, qs2, kv2 = _mlp_qkv(
        x,
        p1["ln2_s"].reshape(1, C), p1["ln2_b"].reshape(1, C),
        p1["w_fc1"], p1["b_fc1"].reshape(1, HID),
        p1["w_fc2"], p1["b_fc2"].reshape(1, C),
        p2["ln1_s"].reshape(1, C), p2["ln1_b"].reshape(1, C),
        p2["w_qkv"], p2["b_qkv"].reshape(1, 3 * C),
    )
    g2 = _gather_pairs(kv2, i1t, GD2).reshape(K, NP, GD2)
    x = _attention2(g2, qs2, rid, x, t2, eh, r, p2["w_proj"], p2["b_proj"].reshape(1, C))
    x = _mlp(
        x,
        p2["ln2_s"].reshape(1, C), p2["ln2_b"].reshape(1, C),
        p2["w_fc1"], p2["b_fc1"].reshape(1, HID),
        p2["w_fc2"], p2["b_fc2"].reshape(1, C),
    )
    return x[:N]


# speedup vs baseline: 1.0000x; 1.0000x over previous
import jax
import jax.numpy as jnp
from jax import lax
from jax.experimental import pallas as pl
from jax.experimental.pallas import tpu as pltpu
from jax.experimental.pallas import tpu_sc as plsc

N = 10000
K = 16
C = 128
H = 8
HC = C // H
WS = 0.16
QS = 0.04
QGL = int((2 * WS + 1e-4) // QS)
L = 2 * QGL
SCALE = HC ** -0.5
HID = 4 * C

NP = 10240
PP = NP * K

GD1 = 3 * C
GD2 = 2 * C


ROWS_A = 1024


def _ln(x, s, b):
    mu = jnp.mean(x, axis=-1, keepdims=True)
    xc = x - mu
    var = jnp.mean(xc * xc, axis=-1, keepdims=True)
    return xc / jnp.sqrt(var + 1e-5) * s + b


def _ln_qkv_body(x_ref, c_ref, s_ref, b_ref, w_ref, bias_ref, qs_ref, kvc_ref):
    h = _ln(x_ref[...], s_ref[...], b_ref[...])
    qkv = jnp.dot(h, w_ref[...], preferred_element_type=jnp.float32) + bias_ref[...]
    qs_ref[...] = qkv[:, :C] * SCALE
    kvc_ref[:, : 2 * C] = qkv[:, C:]
    kvc_ref[:, 2 * C :] = c_ref[...]


def _ln_qkv(feats, coords128, ln_s, ln_b, w_qkv, b_qkv):
    grid = (NP // ROWS_A,)
    return pl.pallas_call(
        _ln_qkv_body,
        grid=grid,
        in_specs=[
            pl.BlockSpec((ROWS_A, C), lambda i: (i, 0)),
            pl.BlockSpec((ROWS_A, C), lambda i: (i, 0)),
            pl.BlockSpec((1, C), lambda i: (0, 0)),
            pl.BlockSpec((1, C), lambda i: (0, 0)),
            pl.BlockSpec((C, 3 * C), lambda i: (0, 0)),
            pl.BlockSpec((1, 3 * C), lambda i: (0, 0)),
        ],
        out_specs=[
            pl.BlockSpec((ROWS_A, C), lambda i: (i, 0)),
            pl.BlockSpec((ROWS_A, GD1), lambda i: (i, 0)),
        ],
        out_shape=[
            jax.ShapeDtypeStruct((NP, C), jnp.float32),
            jax.ShapeDtypeStruct((NP, GD1), jnp.float32),
        ],
    )(feats, coords128, ln_s, ln_b, w_qkv, b_qkv)


CHUNK = 128
NC = 2
NS = 16
NW = NC * NS
NCHUNKS = PP // CHUNK
ITERS = NCHUNKS // NW
OUTER = ITERS // 2


def _gather_pairs(kvc, i1t, width):
    mesh = plsc.VectorSubcoreMesh(
        core_axis_name="c", subcore_axis_name="s", num_cores=NC, num_subcores=NS
    )

    def body(kvc_hbm, idx_hbm, out_hbm, idx_v, rows_v, sem):
        wid = lax.axis_index("s") * NC + lax.axis_index("c")

        def step(j, carry):
            base = (j * NW + wid) * CHUNK
            pltpu.sync_copy(idx_hbm.at[pl.ds(base, CHUNK)], idx_v)
            pltpu.async_copy(kvc_hbm.at[idx_v], rows_v, sem).wait()
            pltpu.sync_copy(rows_v, out_hbm.at[pl.ds(base, CHUNK)])
            return carry

        lax.fori_loop(0, ITERS, step, 0)

    f = pl.kernel(
        body,
        out_type=jax.ShapeDtypeStruct((PP, width), jnp.float32),
        mesh=mesh,
        scratch_types=[
            pltpu.VMEM((CHUNK,), jnp.int32),
            pltpu.VMEM((CHUNK, width), jnp.float32),
            pltpu.SemaphoreType.DMA,
        ],
    )
    return f(kvc, i1t)


TQ = 256
RP = TQ * K
NT = NP // TQ


def _head_matrices():
    lane = jnp.arange(C, dtype=jnp.int32)[:, None]
    head = jnp.arange(H, dtype=jnp.int32)[None, :]
    eh = (lane // HC == head).astype(jnp.float32)
    return eh, eh.T


def _attn_math(kg, vg, rid3, q, resid, t_ref, eh_ref, r_ref, w_ref, b_ref):
    cols = lax.broadcasted_iota(jnp.int32, (K, TQ, 3 * L), 2)
    oh = (
        (cols == (rid3 & (L - 1)))
        | (cols == ((rid3 >> 4) & (L - 1)) + L)
        | (cols == (rid3 >> 8) + 2 * L)
    ).astype(jnp.float32)
    G = jnp.dot(
        oh.reshape(RP, 3 * L), t_ref[...], preferred_element_type=jnp.float32
    ).reshape(K, TQ, 3 * C)

    qg = q[None]
    s = qg * (kg + G[:, :, :C]) + kg * G[:, :, C : 2 * C]
    attn = jnp.dot(
        s.reshape(RP, C), eh_ref[...], preferred_element_type=jnp.float32
    ).reshape(K, TQ, H)

    m = jnp.max(attn, axis=0, keepdims=True)
    e = jnp.exp(attn - m)
    den = jnp.sum(e, axis=0, keepdims=True)
    p = e / den

    pb = jnp.dot(
        p.reshape(RP, H), r_ref[...], preferred_element_type=jnp.float32
    ).reshape(K, TQ, C)
    vt = vg + G[:, :, 2 * C :]
    o = jnp.sum(pb * vt, axis=0)

    x = jnp.dot(o, w_ref[...], preferred_element_type=jnp.float32)
    return x + b_ref[...] + resid


def _attn1_body(g_ref, q_ref, c_ref, res_ref, t_ref, eh_ref, r_ref, w_ref,
                b_ref, o_ref, rid_ref):
    g = g_ref[...]
    cg = g[:, :, 2 * C : 2 * C + 3]
    cq = c_ref[...][None, :, :3]
    rel = cq - cg
    rel = jnp.round(rel * 100000.0) / 100000.0
    rpi = jnp.floor((rel + 2 * WS - 1e-4) / QS).astype(jnp.int32)
    rpi = jnp.clip(rpi, 0, L - 1)
    rid3 = (
        rpi[:, :, 0:1] + rpi[:, :, 1:2] * L + rpi[:, :, 2:3] * (L * L)
    )
    rid_ref[...] = rid3.reshape(1, K, TQ)

    o_ref[...] = _attn_math(
        g[:, :, :C], g[:, :, C : 2 * C], rid3, q_ref[...], res_ref[...],
        t_ref, eh_ref, r_ref, w_ref, b_ref,
    )


def _attn2_body(g_ref, q_ref, rid_ref, res_ref, t_ref, eh_ref, r_ref, w_ref,
                b_ref, o_ref):
    g = g_ref[...]
    rid3 = rid_ref[...].reshape(K, TQ, 1)
    o_ref[...] = _attn_math(
        g[:, :, :C], g[:, :, C : 2 * C], rid3, q_ref[...], res_ref[...],
        t_ref, eh_ref, r_ref, w_ref, b_ref,
    )


def _attention1(g3, qs, coords16, resid, t_stack, eh, r, w_proj, b_proj):
    return pl.pallas_call(
        _attn1_body,
        grid=(NT,),
        in_specs=[
            pl.BlockSpec((K, TQ, GD1), lambda i: (0, i, 0)),
            pl.BlockSpec((TQ, C), lambda i: (i, 0)),
            pl.BlockSpec((TQ, 16), lambda i: (i, 0)),
            pl.BlockSpec((TQ, C), lambda i: (i, 0)),
            pl.BlockSpec((3 * L, 3 * C), lambda i: (0, 0)),
            pl.BlockSpec((C, H), lambda i: (0, 0)),
            pl.BlockSpec((H, C), lambda i: (0, 0)),
            pl.BlockSpec((C, C), lambda i: (0, 0)),
            pl.BlockSpec((1, C), lambda i: (0, 0)),
        ],
        out_specs=[
            pl.BlockSpec((TQ, C), lambda i: (i, 0)),
            pl.BlockSpec((1, K, TQ), lambda i: (i, 0, 0)),
        ],
        out_shape=[
            jax.ShapeDtypeStruct((NP, C), jnp.float32),
            jax.ShapeDtypeStruct((NT, K, TQ), jnp.int32),
        ],
    )(g3, qs, coords16, resid, t_stack, eh, r, w_proj, b_proj)


def _attention2(g3, qs, rid, resid, t_stack, eh, r, w_proj, b_proj):
    return pl.pallas_call(
        _attn2_body,
        grid=(NT,),
        in_specs=[
            pl.BlockSpec((K, TQ, GD2), lambda i: (0, i, 0)),
            pl.BlockSpec((TQ, C), lambda i: (i, 0)),
            pl.BlockSpec((1, K, TQ), lambda i: (i, 0, 0)),
            pl.BlockSpec((TQ, C), lambda i: (i, 0)),
            pl.BlockSpec((3 * L, 3 * C), lambda i: (0, 0)),
            pl.BlockSpec((C, H), lambda i: (0, 0)),
            pl.BlockSpec((H, C), lambda i: (0, 0)),
            pl.BlockSpec((C, C), lambda i: (0, 0)),
            pl.BlockSpec((1, C), lambda i: (0, 0)),
        ],
        out_specs=pl.BlockSpec((TQ, C), lambda i: (i, 0)),
        out_shape=jax.ShapeDtypeStruct((NP, C), jnp.float32),
    )(g3, qs, rid, resid, t_stack, eh, r, w_proj, b_proj)


ROWS_D = 1024


def _mlp_out(x, s_ref, b_ref, w1_ref, b1_ref, w2_ref, b2_ref):
    h = _ln(x, s_ref[...], b_ref[...])
    f = jax.nn.gelu(jnp.dot(h, w1_ref[...], preferred_element_type=jnp.float32) + b1_ref[...])
    return x + jnp.dot(f, w2_ref[...], preferred_element_type=jnp.float32) + b2_ref[...]


def _mlp_body(x_ref, s_ref, b_ref, w1_ref, b1_ref, w2_ref, b2_ref, o_ref):
    o_ref[...] = _mlp_out(x_ref[...], s_ref, b_ref, w1_ref, b1_ref, w2_ref, b2_ref)


def _mlp_qkv_body(x_ref, s_ref, b_ref, w1_ref, b1_ref, w2_ref, b2_ref,
                  s2_ref, bb2_ref, wq_ref, bq_ref, o_ref, qs_ref, kv_ref):
    y = _mlp_out(x_ref[...], s_ref, b_ref, w1_ref, b1_ref, w2_ref, b2_ref)
    o_ref[...] = y
    h = _ln(y, s2_ref[...], bb2_ref[...])
    qkv = jnp.dot(h, wq_ref[...], preferred_element_type=jnp.float32) + bq_ref[...]
    qs_ref[...] = qkv[:, :C] * SCALE
    kv_ref[...] = qkv[:, C:]


def _mlp(x, ln_s, ln_b, w1, b1, w2, b2):
    wspec = [
        pl.BlockSpec((1, C), lambda i: (0, 0)),
        pl.BlockSpec((1, C), lambda i: (0, 0)),
        pl.BlockSpec((C, HID), lambda i: (0, 0)),
        pl.BlockSpec((1, HID), lambda i: (0, 0)),
        pl.BlockSpec((HID, C), lambda i: (0, 0)),
        pl.BlockSpec((1, C), lambda i: (0, 0)),
    ]
    return pl.pallas_call(
        _mlp_body,
        grid=(NP // ROWS_D,),
        in_specs=[pl.BlockSpec((ROWS_D, C), lambda i: (i, 0))] + wspec,
        out_specs=pl.BlockSpec((ROWS_D, C), lambda i: (i, 0)),
        out_shape=jax.ShapeDtypeStruct((NP, C), jnp.float32),
    )(x, ln_s, ln_b, w1, b1, w2, b2)


def _mlp_qkv(x, ln_s, ln_b, w1, b1, w2, b2, ln2_s, ln2_b, w_qkv, b_qkv):
    wspec = [
        pl.BlockSpec((1, C), lambda i: (0, 0)),
        pl.BlockSpec((1, C), lambda i: (0, 0)),
        pl.BlockSpec((C, HID), lambda i: (0, 0)),
        pl.BlockSpec((1, HID), lambda i: (0, 0)),
        pl.BlockSpec((HID, C), lambda i: (0, 0)),
        pl.BlockSpec((1, C), lambda i: (0, 0)),
        pl.BlockSpec((1, C), lambda i: (0, 0)),
        pl.BlockSpec((1, C), lambda i: (0, 0)),
        pl.BlockSpec((C, 3 * C), lambda i: (0, 0)),
        pl.BlockSpec((1, 3 * C), lambda i: (0, 0)),
    ]
    return pl.pallas_call(
        _mlp_qkv_body,
        grid=(NP // ROWS_D,),
        in_specs=[pl.BlockSpec((ROWS_D, C), lambda i: (i, 0))] + wspec,
        out_specs=[
            pl.BlockSpec((ROWS_D, C), lambda i: (i, 0)),
            pl.BlockSpec((ROWS_D, C), lambda i: (i, 0)),
            pl.BlockSpec((ROWS_D, GD2), lambda i: (i, 0)),
        ],
        out_shape=[
            jax.ShapeDtypeStruct((NP, C), jnp.float32),
            jax.ShapeDtypeStruct((NP, C), jnp.float32),
            jax.ShapeDtypeStruct((NP, GD2), jnp.float32),
        ],
    )(x, ln_s, ln_b, w1, b1, w2, b2, ln2_s, ln2_b, w_qkv, b_qkv)


def _stack_tables(p):
    parts = []
    for name in ("tab_q", "tab_k", "tab_v"):
        t = p[name]
        parts.append(jnp.concatenate([t[:, :, :, d].reshape(L, C) for d in range(3)], axis=0))
    return jnp.concatenate(parts, axis=1)


def kernel(feats, coords, params, index_0, index_1, index_0_offsets, n_max):
    feats_p = jnp.pad(feats, ((0, NP - N), (0, 0)))
    coords16 = jnp.pad(coords, ((0, NP - N), (0, 13)))
    coords128 = jnp.pad(coords, ((0, NP - N), (0, C - 3)))
    p1, p2 = params["blocks"]
    t1 = _stack_tables(p1)
    t2 = _stack_tables(p2)
    eh, r = _head_matrices()
    i1t = jnp.pad(index_1.reshape(N, K), ((0, NP - N), (0, 0))).T.reshape(PP)

    qs, kvc = _ln_qkv(
        feats_p, coords128,
        p1["ln1_s"].reshape(1, C), p1["ln1_b"].reshape(1, C),
        p1["w_qkv"], p1["b_qkv"].reshape(1, 3 * C),
    )
    g1 = _gather_pairs(kvc, i1t, GD1).reshape(K, NP, GD1)
    x, rid = _attention1(g1, qs, coords16, feats_p, t1, eh, r, p1["w_proj"], p1["b_proj"].reshape(1, C))
    x, qs2, kv2 = _mlp_qkv(
        x,
        p1["ln2_s"].reshape(1, C), p1["ln2_b"].reshape(1, C),
        p1["w_fc1"], p1["b_fc1"].reshape(1, HID),
        p1["w_fc2"], p1["b_fc2"].reshape(1, C),
        p2["ln1_s"].reshape(1, C), p2["ln1_b"].reshape(1, C),
        p2["w_qkv"], p2["b_qkv"].reshape(1, 3 * C),
    )
    g2 = _gather_pairs(kv2, i1t, GD2).reshape(K, NP, GD2)
    x = _attention2(g2, qs2, rid, x, t2, eh, r, p2["w_proj"], p2["b_proj"].reshape(1, C))
    x = _mlp(
        x,
        p2["ln2_s"].reshape(1, C), p2["ln2_b"].reshape(1, C),
        p2["w_fc1"], p2["b_fc1"].reshape(1, HID),
        p2["w_fc2"], p2["b_fc2"].reshape(1, C),
    )
    return x[:N]


# speedup vs baseline: 1.4671x; 1.4671x over previous
import jax
import jax.numpy as jnp
from jax import lax
from jax.experimental import pallas as pl
from jax.experimental.pallas import tpu as pltpu
from jax.experimental.pallas import tpu_sc as plsc

N = 10000
K = 16
C = 128
H = 8
HC = C // H
WS = 0.16
QS = 0.04
QGL = int((2 * WS + 1e-4) // QS)
L = 2 * QGL
SCALE = HC ** -0.5
HID = 4 * C

NP = N
PP = NP * K

GD1 = 3 * C
GD2 = 2 * C


ROWS_A = 1000


def _ln(x, s, b):
    mu = jnp.mean(x, axis=-1, keepdims=True)
    xc = x - mu
    var = jnp.mean(xc * xc, axis=-1, keepdims=True)
    return xc / jnp.sqrt(var + 1e-5) * s + b


def _ln_qkv_body(x_ref, c_ref, s_ref, b_ref, w_ref, bias_ref, qs_ref, kvc_ref):
    h = _ln(x_ref[...], s_ref[...], b_ref[...])
    qkv = jnp.dot(h, w_ref[...], preferred_element_type=jnp.float32) + bias_ref[...]
    qs_ref[...] = qkv[:, :C] * SCALE
    kvc_ref[:, : 2 * C] = qkv[:, C:]
    kvc_ref[:, 2 * C :] = c_ref[...]


def _ln_qkv(feats, coords128, ln_s, ln_b, w_qkv, b_qkv):
    grid = (NP // ROWS_A,)
    return pl.pallas_call(
        _ln_qkv_body,
        grid=grid,
        in_specs=[
            pl.BlockSpec((ROWS_A, C), lambda i: (i, 0)),
            pl.BlockSpec((ROWS_A, C), lambda i: (i, 0)),
            pl.BlockSpec((1, C), lambda i: (0, 0)),
            pl.BlockSpec((1, C), lambda i: (0, 0)),
            pl.BlockSpec((C, 3 * C), lambda i: (0, 0)),
            pl.BlockSpec((1, 3 * C), lambda i: (0, 0)),
        ],
        out_specs=[
            pl.BlockSpec((ROWS_A, C), lambda i: (i, 0)),
            pl.BlockSpec((ROWS_A, GD1), lambda i: (i, 0)),
        ],
        out_shape=[
            jax.ShapeDtypeStruct((NP, C), jnp.float32),
            jax.ShapeDtypeStruct((NP, GD1), jnp.float32),
        ],
    )(feats, coords128, ln_s, ln_b, w_qkv, b_qkv)


CHUNK = 128
NC = 2
NS = 16
NW = NC * NS
NCHUNKS = PP // CHUNK
ITERS = (NCHUNKS + NW - 1) // NW


def _gather_pairs(kv, i1t, width):
    mesh = plsc.VectorSubcoreMesh(
        core_axis_name="c", subcore_axis_name="s", num_cores=NC, num_subcores=NS
    )

    def body(kv_hbm, idx_hbm, out_hbm, idx_v, rows_v, sem):
        wid = lax.axis_index("s") * NC + lax.axis_index("c")

        def step(j, carry):
            chunk = j * NW + wid

            @pl.when(chunk < NCHUNKS)
            def _():
                base = chunk * CHUNK
                pltpu.sync_copy(idx_hbm.at[pl.ds(base, CHUNK)], idx_v)
                pltpu.async_copy(kv_hbm.at[idx_v], rows_v, sem).wait()
                pltpu.sync_copy(rows_v, out_hbm.at[pl.ds(base, CHUNK)])

            return carry

        lax.fori_loop(0, ITERS, step, 0)

    f = pl.kernel(
        body,
        out_type=jax.ShapeDtypeStruct((PP, width), jnp.float32),
        mesh=mesh,
        scratch_types=[
            pltpu.VMEM((CHUNK,), jnp.int32),
            pltpu.VMEM((CHUNK, width), jnp.float32),
            pltpu.SemaphoreType.DMA,
        ],
    )
    return f(kv, i1t)


TQ = 200
RP = TQ * K
NT = NP // TQ


def _head_matrices():
    lane = jnp.arange(C, dtype=jnp.int32)[:, None]
    head = jnp.arange(H, dtype=jnp.int32)[None, :]
    eh = (lane // HC == head).astype(jnp.float32)
    return eh, eh.T


def _attn_math(kg, vg, rid3, q, resid, t_ref, eh_ref, r_ref, w_ref, b_ref):
    cols = lax.broadcasted_iota(jnp.int32, (K, TQ, 3 * L), 2)
    oh = (
        (cols == (rid3 & (L - 1)))
        | (cols == ((rid3 >> 4) & (L - 1)) + L)
        | (cols == (rid3 >> 8) + 2 * L)
    ).astype(jnp.float32)
    G = jnp.dot(
        oh.reshape(RP, 3 * L), t_ref[...], preferred_element_type=jnp.float32
    ).reshape(K, TQ, 3 * C)

    qg = q[None]
    s = qg * (kg + G[:, :, :C]) + kg * G[:, :, C : 2 * C]
    attn = jnp.dot(
        s.reshape(RP, C), eh_ref[...], preferred_element_type=jnp.float32
    ).reshape(K, TQ, H)

    m = jnp.max(attn, axis=0, keepdims=True)
    e = jnp.exp(attn - m)
    den = jnp.sum(e, axis=0, keepdims=True)
    p = e / den

    pb = jnp.dot(
        p.reshape(RP, H), r_ref[...], preferred_element_type=jnp.float32
    ).reshape(K, TQ, C)
    vt = vg + G[:, :, 2 * C :]
    o = jnp.sum(pb * vt, axis=0)

    x = jnp.dot(o, w_ref[...], preferred_element_type=jnp.float32)
    return x + b_ref[...] + resid


def _mlp_out(x, s_ref, b_ref, w1_ref, b1_ref, w2_ref, b2_ref):
    h = _ln(x, s_ref[...], b_ref[...])
    f = jax.nn.gelu(jnp.dot(h, w1_ref[...], preferred_element_type=jnp.float32) + b1_ref[...])
    return x + jnp.dot(f, w2_ref[...], preferred_element_type=jnp.float32) + b2_ref[...]


_COMMON_SPECS = [
    pl.BlockSpec((TQ, C), lambda i: (i, 0)),
    pl.BlockSpec((3 * L, 3 * C), lambda i: (0, 0)),
    pl.BlockSpec((C, H), lambda i: (0, 0)),
    pl.BlockSpec((H, C), lambda i: (0, 0)),
    pl.BlockSpec((C, C), lambda i: (0, 0)),
    pl.BlockSpec((1, C), lambda i: (0, 0)),
]
_MLP_SPECS = [
    pl.BlockSpec((1, C), lambda i: (0, 0)),
    pl.BlockSpec((1, C), lambda i: (0, 0)),
    pl.BlockSpec((C, HID), lambda i: (0, 0)),
    pl.BlockSpec((1, HID), lambda i: (0, 0)),
    pl.BlockSpec((HID, C), lambda i: (0, 0)),
    pl.BlockSpec((1, C), lambda i: (0, 0)),
]
_QKV_SPECS = [
    pl.BlockSpec((1, C), lambda i: (0, 0)),
    pl.BlockSpec((1, C), lambda i: (0, 0)),
    pl.BlockSpec((C, 3 * C), lambda i: (0, 0)),
    pl.BlockSpec((1, 3 * C), lambda i: (0, 0)),
]


def _rid_from_coords(g, cq):
    cg = g[:, :, 2 * C : 2 * C + 3]
    rel = cq[None, :, :3] - cg
    rel = jnp.round(rel * 100000.0) / 100000.0
    rpi = jnp.floor((rel + 2 * WS - 1e-4) / QS).astype(jnp.int32)
    rpi = jnp.clip(rpi, 0, L - 1)
    return rpi[:, :, 0:1] + rpi[:, :, 1:2] * L + rpi[:, :, 2:3] * (L * L)


def _attn_mlp_body(g_ref, q_ref, rid_ref, res_ref, t_ref, eh_ref, r_ref,
                   w_ref, b_ref, s_ref, bb_ref, w1_ref, b1_ref, w2_ref,
                   b2_ref, o_ref):
    g = g_ref[...]
    rid3 = rid_ref[...].reshape(K, TQ, 1)
    x = _attn_math(
        g[:, :, :C], g[:, :, C : 2 * C], rid3, q_ref[...], res_ref[...],
        t_ref, eh_ref, r_ref, w_ref, b_ref,
    )
    o_ref[...] = _mlp_out(x, s_ref, bb_ref, w1_ref, b1_ref, w2_ref, b2_ref)


def _attn_mlp_qkv_body(g_ref, q_ref, c_ref, res_ref, t_ref, eh_ref, r_ref,
                       w_ref, b_ref, s_ref, bb_ref, w1_ref, b1_ref, w2_ref,
                       b2_ref, s2_ref, bb2_ref, wq_ref, bq_ref,
                       o_ref, qs_ref, kv_ref, rid_ref):
    g = g_ref[...]
    rid3 = _rid_from_coords(g, c_ref[...])
    rid_ref[...] = rid3.reshape(1, K, TQ)
    x = _attn_math(
        g[:, :, :C], g[:, :, C : 2 * C], rid3, q_ref[...], res_ref[...],
        t_ref, eh_ref, r_ref, w_ref, b_ref,
    )
    y = _mlp_out(x, s_ref, bb_ref, w1_ref, b1_ref, w2_ref, b2_ref)
    o_ref[...] = y
    h = _ln(y, s2_ref[...], bb2_ref[...])
    qkv = jnp.dot(h, wq_ref[...], preferred_element_type=jnp.float32) + bq_ref[...]
    qs_ref[...] = qkv[:, :C] * SCALE
    kv_ref[...] = qkv[:, C:]


def _attention_mlp(g3, qs, rid3d, resid, t_stack, eh, r, w_proj, b_proj,
                   ln_s, ln_b, w1, b1, w2, b2):
    return pl.pallas_call(
        _attn_mlp_body,
        grid=(NT,),
        in_specs=[
            pl.BlockSpec((K, TQ, GD2), lambda i: (0, i, 0)),
            pl.BlockSpec((TQ, C), lambda i: (i, 0)),
            pl.BlockSpec((1, K, TQ), lambda i: (i, 0, 0)),
        ] + _COMMON_SPECS + _MLP_SPECS,
        out_specs=pl.BlockSpec((TQ, C), lambda i: (i, 0)),
        out_shape=jax.ShapeDtypeStruct((NP, C), jnp.float32),
    )(g3, qs, rid3d, resid, t_stack, eh, r, w_proj, b_proj,
      ln_s, ln_b, w1, b1, w2, b2)


def _attention_mlp_qkv(g3, qs, coords16, resid, t_stack, eh, r, w_proj,
                       b_proj, ln_s, ln_b, w1, b1, w2, b2, ln2_s, ln2_b,
                       w_qkv, b_qkv):
    return pl.pallas_call(
        _attn_mlp_qkv_body,
        grid=(NT,),
        in_specs=[
            pl.BlockSpec((K, TQ, GD1), lambda i: (0, i, 0)),
            pl.BlockSpec((TQ, C), lambda i: (i, 0)),
            pl.BlockSpec((TQ, 16), lambda i: (i, 0)),
        ] + _COMMON_SPECS + _MLP_SPECS + _QKV_SPECS,
        out_specs=[
            pl.BlockSpec((TQ, C), lambda i: (i, 0)),
            pl.BlockSpec((TQ, C), lambda i: (i, 0)),
            pl.BlockSpec((TQ, GD2), lambda i: (i, 0)),
            pl.BlockSpec((1, K, TQ), lambda i: (i, 0, 0)),
        ],
        out_shape=[
            jax.ShapeDtypeStruct((NP, C), jnp.float32),
            jax.ShapeDtypeStruct((NP, C), jnp.float32),
            jax.ShapeDtypeStruct((NP, GD2), jnp.float32),
            jax.ShapeDtypeStruct((NT, K, TQ), jnp.int32),
        ],
    )(g3, qs, coords16, resid, t_stack, eh, r, w_proj, b_proj,
      ln_s, ln_b, w1, b1, w2, b2, ln2_s, ln2_b, w_qkv, b_qkv)


def _stack_tables(p):
    parts = []
    for name in ("tab_q", "tab_k", "tab_v"):
        t = p[name]
        parts.append(jnp.concatenate([t[:, :, :, d].reshape(L, C) for d in range(3)], axis=0))
    return jnp.concatenate(parts, axis=1)


def kernel(feats, coords, params, index_0, index_1, index_0_offsets, n_max):
    p1, p2 = params["blocks"]
    t1 = _stack_tables(p1)
    t2 = _stack_tables(p2)
    eh, r = _head_matrices()
    coords16 = jnp.pad(coords, ((0, 0), (0, 13)))
    coords128 = jnp.pad(coords, ((0, 0), (0, C - 3)))
    i1t = index_1.reshape(N, K).T.reshape(PP)

    qs, kvc = _ln_qkv(
        feats, coords128,
        p1["ln1_s"].reshape(1, C), p1["ln1_b"].reshape(1, C),
        p1["w_qkv"], p1["b_qkv"].reshape(1, 3 * C),
    )
    g1 = _gather_pairs(kvc, i1t, GD1).reshape(K, NP, GD1)
    x, qs2, kv2, rid = _attention_mlp_qkv(
        g1, qs, coords16, feats, t1, eh, r,
        p1["w_proj"], p1["b_proj"].reshape(1, C),
        p1["ln2_s"].reshape(1, C), p1["ln2_b"].reshape(1, C),
        p1["w_fc1"], p1["b_fc1"].reshape(1, HID),
        p1["w_fc2"], p1["b_fc2"].reshape(1, C),
        p2["ln1_s"].reshape(1, C), p2["ln1_b"].reshape(1, C),
        p2["w_qkv"], p2["b_qkv"].reshape(1, 3 * C),
    )
    g2 = _gather_pairs(kv2, i1t, GD2).reshape(K, NP, GD2)
    x = _attention_mlp(
        g2, qs2, rid, x, t2, eh, r,
        p2["w_proj"], p2["b_proj"].reshape(1, C),
        p2["ln2_s"].reshape(1, C), p2["ln2_b"].reshape(1, C),
        p2["w_fc1"], p2["b_fc1"].reshape(1, HID),
        p2["w_fc2"], p2["b_fc2"].reshape(1, C),
    )
    return x


# speedup vs baseline: 1.5205x; 1.0364x over previous
import jax
import jax.numpy as jnp
from jax import lax
from jax.experimental import pallas as pl
from jax.experimental.pallas import tpu as pltpu
from jax.experimental.pallas import tpu_sc as plsc

N = 10000
K = 16
C = 128
H = 8
HC = C // H
WS = 0.16
QS = 0.04
QGL = int((2 * WS + 1e-4) // QS)
L = 2 * QGL
SCALE = HC ** -0.5
HID = 4 * C

NP = N
PP = NP * K

GD1 = 3 * C
GD2 = 2 * C


ROWS_A = 1000


def _ln(x, s, b):
    mu = jnp.mean(x, axis=-1, keepdims=True)
    xc = x - mu
    var = jnp.mean(xc * xc, axis=-1, keepdims=True)
    return xc / jnp.sqrt(var + 1e-5) * s + b


def _ln_qkv_body(x_ref, c_ref, s_ref, b_ref, w_ref, bias_ref, qs_ref, kvc_ref):
    h = _ln(x_ref[...], s_ref[...], b_ref[...])
    qkv = jnp.dot(h, w_ref[...], preferred_element_type=jnp.float32) + bias_ref[...]
    qs_ref[...] = qkv[:, :C] * SCALE
    kvc_ref[:, : 2 * C] = qkv[:, C:]
    kvc_ref[:, 2 * C :] = c_ref[...]


def _ln_qkv(feats, coords128, ln_s, ln_b, w_qkv, b_qkv):
    grid = (NP // ROWS_A,)
    return pl.pallas_call(
        _ln_qkv_body,
        grid=grid,
        in_specs=[
            pl.BlockSpec((ROWS_A, C), lambda i: (i, 0)),
            pl.BlockSpec((ROWS_A, C), lambda i: (i, 0)),
            pl.BlockSpec((1, C), lambda i: (0, 0)),
            pl.BlockSpec((1, C), lambda i: (0, 0)),
            pl.BlockSpec((C, 3 * C), lambda i: (0, 0)),
            pl.BlockSpec((1, 3 * C), lambda i: (0, 0)),
        ],
        out_specs=[
            pl.BlockSpec((ROWS_A, C), lambda i: (i, 0)),
            pl.BlockSpec((ROWS_A, GD1), lambda i: (i, 0)),
        ],
        out_shape=[
            jax.ShapeDtypeStruct((NP, C), jnp.float32),
            jax.ShapeDtypeStruct((NP, GD1), jnp.float32),
        ],
    )(feats, coords128, ln_s, ln_b, w_qkv, b_qkv)


CHUNK = 128
NC = 2
NS = 16
NW = NC * NS
NCHUNKS = PP // CHUNK
ITERS = (NCHUNKS + NW - 1) // NW


def _gather_pairs(kv, i1t, width):
    mesh = plsc.VectorSubcoreMesh(
        core_axis_name="c", subcore_axis_name="s", num_cores=NC, num_subcores=NS
    )

    def body(kv_hbm, idx_hbm, out_hbm, idx_v, rows_v, sem):
        wid = lax.axis_index("s") * NC + lax.axis_index("c")

        def step(j, carry):
            chunk = j * NW + wid

            @pl.when(chunk < NCHUNKS)
            def _():
                base = chunk * CHUNK
                pltpu.sync_copy(idx_hbm.at[pl.ds(base, CHUNK)], idx_v)
                pltpu.async_copy(kv_hbm.at[idx_v], rows_v, sem).wait()
                pltpu.sync_copy(rows_v, out_hbm.at[pl.ds(base, CHUNK)])

            return carry

        lax.fori_loop(0, ITERS, step, 0)

    f = pl.kernel(
        body,
        out_type=jax.ShapeDtypeStruct((PP, width), jnp.float32),
        mesh=mesh,
        scratch_types=[
            pltpu.VMEM((CHUNK,), jnp.int32),
            pltpu.VMEM((CHUNK, width), jnp.float32),
            pltpu.SemaphoreType.DMA,
        ],
    )
    return f(kv, i1t)


TQ = 200
RP = TQ * K
NT = NP // TQ


def _head_matrices():
    lane = jnp.arange(C, dtype=jnp.int32)[:, None]
    head = jnp.arange(H, dtype=jnp.int32)[None, :]
    eh = (lane // HC == head).astype(jnp.float32)
    return eh, eh.T


def _attn_math(kg, vg, rid3, q, resid, t_ref, eh_ref, r_ref, w_ref, b_ref):
    cols = lax.broadcasted_iota(jnp.int32, (K, TQ, 3 * L), 2)
    oh = (
        (cols == (rid3 & (L - 1)))
        | (cols == ((rid3 >> 4) & (L - 1)) + L)
        | (cols == (rid3 >> 8) + 2 * L)
    ).astype(jnp.float32)
    G = jnp.dot(
        oh.reshape(RP, 3 * L), t_ref[...], preferred_element_type=jnp.float32
    ).reshape(K, TQ, 3 * C)

    qg = q[None]
    s = qg * (kg + G[:, :, :C]) + kg * G[:, :, C : 2 * C]
    attn = jnp.dot(
        s.reshape(RP, C), eh_ref[...], preferred_element_type=jnp.float32
    ).reshape(K, TQ, H)

    m = jnp.max(attn, axis=0, keepdims=True)
    e = jnp.exp(attn - m)
    den = jnp.sum(e, axis=0, keepdims=True)
    p = e / den

    pb = jnp.dot(
        p.reshape(RP, H), r_ref[...], preferred_element_type=jnp.float32
    ).reshape(K, TQ, C)
    vt = vg + G[:, :, 2 * C :]
    o = jnp.sum(pb * vt, axis=0)

    x = jnp.dot(o, w_ref[...], preferred_element_type=jnp.float32)
    return x + b_ref[...] + resid


def _mlp_out(x, s_ref, b_ref, w1_ref, b1_ref, w2_ref, b2_ref):
    h = _ln(x, s_ref[...], b_ref[...])
    f = jax.nn.gelu(jnp.dot(h, w1_ref[...], preferred_element_type=jnp.float32) + b1_ref[...])
    return x + jnp.dot(f, w2_ref[...], preferred_element_type=jnp.float32) + b2_ref[...]


_COMMON_SPECS = [
    pl.BlockSpec((TQ, C), lambda i: (i, 0)),
    pl.BlockSpec((3 * L, 3 * C), lambda i: (0, 0)),
    pl.BlockSpec((C, H), lambda i: (0, 0)),
    pl.BlockSpec((H, C), lambda i: (0, 0)),
    pl.BlockSpec((C, C), lambda i: (0, 0)),
    pl.BlockSpec((1, C), lambda i: (0, 0)),
]
_MLP_SPECS = [
    pl.BlockSpec((1, C), lambda i: (0, 0)),
    pl.BlockSpec((1, C), lambda i: (0, 0)),
    pl.BlockSpec((C, HID), lambda i: (0, 0)),
    pl.BlockSpec((1, HID), lambda i: (0, 0)),
    pl.BlockSpec((HID, C), lambda i: (0, 0)),
    pl.BlockSpec((1, C), lambda i: (0, 0)),
]
_QKV_SPECS = [
    pl.BlockSpec((1, C), lambda i: (0, 0)),
    pl.BlockSpec((1, C), lambda i: (0, 0)),
    pl.BlockSpec((C, 3 * C), lambda i: (0, 0)),
    pl.BlockSpec((1, 3 * C), lambda i: (0, 0)),
]


def _rid_from_coords(g, cq):
    cg = g[:, :, 2 * C : 2 * C + 3]
    rel = cq[None, :, :3] - cg
    rel = jnp.round(rel * 100000.0) / 100000.0
    rpi = jnp.floor((rel + 2 * WS - 1e-4) / QS).astype(jnp.int32)
    rpi = jnp.clip(rpi, 0, L - 1)
    return rpi[:, :, 0:1] + rpi[:, :, 1:2] * L + rpi[:, :, 2:3] * (L * L)


def _attn1_body(g_ref, q_ref, c_ref, res_ref, t_ref, eh_ref, r_ref,
                w_ref, b_ref, o_ref, rid_ref):
    g = g_ref[...]
    rid3 = _rid_from_coords(g, c_ref[...])
    rid_ref[...] = rid3.reshape(1, K, TQ)
    o_ref[...] = _attn_math(
        g[:, :, :C], g[:, :, C : 2 * C], rid3, q_ref[...], res_ref[...],
        t_ref, eh_ref, r_ref, w_ref, b_ref,
    )


def _attn2_body(g_ref, q_ref, rid_ref, res_ref, t_ref, eh_ref, r_ref,
                w_ref, b_ref, o_ref):
    g = g_ref[...]
    rid3 = rid_ref[...].reshape(K, TQ, 1)
    o_ref[...] = _attn_math(
        g[:, :, :C], g[:, :, C : 2 * C], rid3, q_ref[...], res_ref[...],
        t_ref, eh_ref, r_ref, w_ref, b_ref,
    )


def _attention1(g3, qs, coords16, resid, t_stack, eh, r, w_proj, b_proj):
    return pl.pallas_call(
        _attn1_body,
        grid=(NT,),
        in_specs=[
            pl.BlockSpec((K, TQ, GD1), lambda i: (0, i, 0)),
            pl.BlockSpec((TQ, C), lambda i: (i, 0)),
            pl.BlockSpec((TQ, 16), lambda i: (i, 0)),
        ] + _COMMON_SPECS,
        out_specs=[
            pl.BlockSpec((TQ, C), lambda i: (i, 0)),
            pl.BlockSpec((1, K, TQ), lambda i: (i, 0, 0)),
        ],
        out_shape=[
            jax.ShapeDtypeStruct((NP, C), jnp.float32),
            jax.ShapeDtypeStruct((NT, K, TQ), jnp.int32),
        ],
    )(g3, qs, coords16, resid, t_stack, eh, r, w_proj, b_proj)


def _attention2(g3, qs, rid3d, resid, t_stack, eh, r, w_proj, b_proj):
    return pl.pallas_call(
        _attn2_body,
        grid=(NT,),
        in_specs=[
            pl.BlockSpec((K, TQ, GD2), lambda i: (0, i, 0)),
            pl.BlockSpec((TQ, C), lambda i: (i, 0)),
            pl.BlockSpec((1, K, TQ), lambda i: (i, 0, 0)),
        ] + _COMMON_SPECS,
        out_specs=pl.BlockSpec((TQ, C), lambda i: (i, 0)),
        out_shape=jax.ShapeDtypeStruct((NP, C), jnp.float32),
    )(g3, qs, rid3d, resid, t_stack, eh, r, w_proj, b_proj)


ROWS_D = 1000


def _mlp_body(x_ref, s_ref, b_ref, w1_ref, b1_ref, w2_ref, b2_ref, o_ref):
    o_ref[...] = _mlp_out(x_ref[...], s_ref, b_ref, w1_ref, b1_ref, w2_ref, b2_ref)


def _mlp_qkv_body(x_ref, s_ref, b_ref, w1_ref, b1_ref, w2_ref, b2_ref,
                  s2_ref, bb2_ref, wq_ref, bq_ref, o_ref, qs_ref, kv_ref):
    y = _mlp_out(x_ref[...], s_ref, b_ref, w1_ref, b1_ref, w2_ref, b2_ref)
    o_ref[...] = y
    h = _ln(y, s2_ref[...], bb2_ref[...])
    qkv = jnp.dot(h, wq_ref[...], preferred_element_type=jnp.float32) + bq_ref[...]
    qs_ref[...] = qkv[:, :C] * SCALE
    kv_ref[...] = qkv[:, C:]


def _mlp(x, ln_s, ln_b, w1, b1, w2, b2):
    return pl.pallas_call(
        _mlp_body,
        grid=(NP // ROWS_D,),
        in_specs=[pl.BlockSpec((ROWS_D, C), lambda i: (i, 0))] + _MLP_SPECS,
        out_specs=pl.BlockSpec((ROWS_D, C), lambda i: (i, 0)),
        out_shape=jax.ShapeDtypeStruct((NP, C), jnp.float32),
    )(x, ln_s, ln_b, w1, b1, w2, b2)


def _mlp_qkv(x, ln_s, ln_b, w1, b1, w2, b2, ln2_s, ln2_b, w_qkv, b_qkv):
    return pl.pallas_call(
        _mlp_qkv_body,
        grid=(NP // ROWS_D,),
        in_specs=[pl.BlockSpec((ROWS_D, C), lambda i: (i, 0))]
        + _MLP_SPECS + _QKV_SPECS,
        out_specs=[
            pl.BlockSpec((ROWS_D, C), lambda i: (i, 0)),
            pl.BlockSpec((ROWS_D, C), lambda i: (i, 0)),
            pl.BlockSpec((ROWS_D, GD2), lambda i: (i, 0)),
        ],
        out_shape=[
            jax.ShapeDtypeStruct((NP, C), jnp.float32),
            jax.ShapeDtypeStruct((NP, C), jnp.float32),
            jax.ShapeDtypeStruct((NP, GD2), jnp.float32),
        ],
    )(x, ln_s, ln_b, w1, b1, w2, b2, ln2_s, ln2_b, w_qkv, b_qkv)


def _stack_tables(p):
    parts = []
    for name in ("tab_q", "tab_k", "tab_v"):
        t = p[name]
        parts.append(jnp.concatenate([t[:, :, :, d].reshape(L, C) for d in range(3)], axis=0))
    return jnp.concatenate(parts, axis=1)


def kernel(feats, coords, params, index_0, index_1, index_0_offsets, n_max):
    p1, p2 = params["blocks"]
    t1 = _stack_tables(p1)
    t2 = _stack_tables(p2)
    eh, r = _head_matrices()
    coords16 = jnp.pad(coords, ((0, 0), (0, 13)))
    coords128 = jnp.pad(coords, ((0, 0), (0, C - 3)))
    i1t = index_1.reshape(N, K).T.reshape(PP)

    qs, kvc = _ln_qkv(
        feats, coords128,
        p1["ln1_s"].reshape(1, C), p1["ln1_b"].reshape(1, C),
        p1["w_qkv"], p1["b_qkv"].reshape(1, 3 * C),
    )
    g1 = _gather_pairs(kvc, i1t, GD1).reshape(K, NP, GD1)
    x, rid = _attention1(
        g1, qs, coords16, feats, t1, eh, r,
        p1["w_proj"], p1["b_proj"].reshape(1, C),
    )
    x, qs2, kv2 = _mlp_qkv(
        x,
        p1["ln2_s"].reshape(1, C), p1["ln2_b"].reshape(1, C),
        p1["w_fc1"], p1["b_fc1"].reshape(1, HID),
        p1["w_fc2"], p1["b_fc2"].reshape(1, C),
        p2["ln1_s"].reshape(1, C), p2["ln1_b"].reshape(1, C),
        p2["w_qkv"], p2["b_qkv"].reshape(1, 3 * C),
    )
    g2 = _gather_pairs(kv2, i1t, GD2).reshape(K, NP, GD2)
    x = _attention2(
        g2, qs2, rid, x, t2, eh, r,
        p2["w_proj"], p2["b_proj"].reshape(1, C),
    )
    x = _mlp(
        x,
        p2["ln2_s"].reshape(1, C), p2["ln2_b"].reshape(1, C),
        p2["w_fc1"], p2["b_fc1"].reshape(1, HID),
        p2["w_fc2"], p2["b_fc2"].reshape(1, C),
    )
    return x
